# baseline (speedup 1.0000x reference)
import functools

import jax
import jax.numpy as jnp
from jax import lax
from jax.experimental import pallas as pl
from jax.experimental.pallas import tpu as pltpu
from jax.experimental.pallas import tpu_sc as plsc

_K = 20
_D = 64
_NC = 2
_NS = 16
_NW = _NC * _NS
_L = 16
_ISL = 128


@functools.lru_cache(maxsize=None)
def _make_sc_scores(B, interpret=False):
    PW = B // _NW
    CH = min(64, PW)
    NCH = PW // CH
    NEG_CH = CH * _K
    NSL = NEG_CH // _ISL
    assert B % _NW == 0 and PW % CH == 0 and NEG_CH % _ISL == 0

    mesh = plsc.VectorSubcoreMesh(core_axis_name="c", subcore_axis_name="s")

    @functools.partial(
        pl.kernel,
        out_type=(jax.ShapeDtypeStruct((B,), jnp.float32),
                  jax.ShapeDtypeStruct((B * _K,), jnp.float32)),
        mesh=mesh,
        interpret=interpret,
        compiler_params=pltpu.CompilerParams(
            needs_layout_passes=False, use_tc_tiling_on_sc=False),
        scratch_types=[
            pltpu.VMEM((CH,), jnp.int32),
            pltpu.VMEM((CH,), jnp.int32),
            pltpu.VMEM((NSL, _ISL), jnp.int32),
            pltpu.VMEM((CH, _D), jnp.float32),
            pltpu.VMEM((CH, _D), jnp.float32),
            pltpu.VMEM((NEG_CH, _D), jnp.float32),
            pltpu.VMEM((CH,), jnp.float32),
            pltpu.VMEM((NEG_CH,), jnp.float32),
            pltpu.SemaphoreType.DMA,
        ],
    )
    def sgns_scores(centers_h, contexts_h, negs_h, inemb_h, outemb_h,
                    pos_h, neg_h, cidx, oidx, nidx, vc, uo, ne, posv, negv,
                    sem):
        wid = lax.axis_index("s") * _NC + lax.axis_index("c")
        lanes = lax.iota(jnp.int32, _L)
        for ch in range(NCH):
            base = wid * PW + ch * CH
            nbase = base * _K
            pltpu.sync_copy(centers_h.at[pl.ds(base, CH)], cidx)
            pltpu.sync_copy(contexts_h.at[pl.ds(base, CH)], oidx)
            for j in range(NSL):
                pltpu.sync_copy(negs_h.at[pl.ds(nbase + j * _ISL, _ISL)],
                                nidx.at[j])
            cps = [pltpu.async_copy(inemb_h.at[cidx], vc, sem),
                   pltpu.async_copy(outemb_h.at[oidx], uo, sem)]
            for j in range(NSL):
                cps.append(pltpu.async_copy(outemb_h.at[nidx.at[j]],
                                            ne.at[pl.ds(j * _ISL, _ISL)],
                                            sem))
            for cp in cps:
                cp.wait()
            for g in range(CH // _L):
                rows = g * _L + lanes
                nrows0 = rows * _K

                def dbody(d, accs, rows=rows, nrows0=nrows0):
                    pos_acc, negaccs = accs
                    dcol = jnp.broadcast_to(d, (_L,))
                    vcd = plsc.load_gather(vc, [rows, dcol])
                    uod = plsc.load_gather(uo, [rows, dcol])
                    pos_acc = pos_acc + vcd * uod
                    new = tuple(
                        negaccs[k]
                        + plsc.load_gather(ne, [nrows0 + k, dcol]) * vcd
                        for k in range(_K))
                    return pos_acc, new

                zero = jnp.zeros((_L,), jnp.float32)
                pos_acc, negaccs = lax.fori_loop(
                    0, _D, dbody, (zero, tuple(zero for _ in range(_K))))
                posv[pl.ds(g * _L, _L)] = pos_acc
                for k in range(_K):
                    negv[pl.ds(k * CH + g * _L, _L)] = negaccs[k]
            pltpu.sync_copy(posv, pos_h.at[pl.ds(base, CH)])
            pltpu.sync_copy(negv, neg_h.at[pl.ds(nbase, NEG_CH)])

    return sgns_scores


@functools.lru_cache(maxsize=None)
def _make_tc_loss(B, interpret=False):
    def body(pos_ref, neg_ref, out_ref):
        p = pos_ref[...]
        n = neg_ref[...]
        lp = -jnp.log(jax.nn.sigmoid(p) + 1e-10)
        ln = -jnp.log(jax.nn.sigmoid(-n) + 1e-10)
        out_ref[...] = ((jnp.sum(lp) + jnp.sum(ln)) / B).reshape(1, 1)

    return pl.pallas_call(
        body,
        out_shape=jax.ShapeDtypeStruct((1, 1), jnp.float32),
        interpret=interpret,
    )


def _sgns(centers, contexts, negs, in_embed, out_embed, interpret=False):
    B = centers.shape[0]
    c = centers.reshape(-1).astype(jnp.int32)
    o = contexts.reshape(-1).astype(jnp.int32)
    n = negs.reshape(-1).astype(jnp.int32)
    pos_s, neg_s = _make_sc_scores(B, interpret)(c, o, n, in_embed, out_embed)
    loss = _make_tc_loss(B, interpret)(
        pos_s.reshape(B // 128, 128), neg_s.reshape(B * _K // 128, 128))
    return loss[0, 0]


def kernel(centers, contexts, negs, in_embed, out_embed):
    return _sgns(centers, contexts, negs, in_embed, out_embed)


# speedup vs baseline: 1.0551x; 1.0551x over previous
import functools

import jax
import jax.numpy as jnp
from jax import lax
from jax.experimental import pallas as pl
from jax.experimental.pallas import tpu as pltpu
from jax.experimental.pallas import tpu_sc as plsc

_K = 20
_D = 64
_NC = 2
_NS = 16
_NW = _NC * _NS
_L = 16
_DB = 16


@functools.lru_cache(maxsize=None)
def _make_sc_scores(B):
    PW = B // _NW
    CH = min(_L, PW)
    NCH = PW // CH
    NEG_CH = CH * _K
    NSL = 4
    ISL = NEG_CH // NSL
    NPW = PW * _K
    assert B % _NW == 0 and PW % CH == 0 and NEG_CH % NSL == 0
    assert ISL <= 128 and ISL % 8 == 0 and NCH % 2 == 0

    mesh = plsc.VectorSubcoreMesh(core_axis_name="c", subcore_axis_name="s")

    @functools.partial(
        pl.kernel,
        out_type=(jax.ShapeDtypeStruct((B,), jnp.float32),
                  jax.ShapeDtypeStruct((B * _K,), jnp.float32)),
        mesh=mesh,
        compiler_params=pltpu.CompilerParams(
            needs_layout_passes=False, use_tc_tiling_on_sc=False),
        scratch_types=[
            pltpu.VMEM((PW,), jnp.int32),
            pltpu.VMEM((PW,), jnp.int32),
            pltpu.VMEM((NPW,), jnp.int32),
            pltpu.VMEM((2, CH, _D), jnp.float32),
            pltpu.VMEM((2, CH, _D), jnp.float32),
            pltpu.VMEM((2, NEG_CH, _D), jnp.float32),
            pltpu.VMEM((PW,), jnp.float32),
            pltpu.VMEM((NPW,), jnp.float32),
            pltpu.SemaphoreType.DMA,
            pltpu.SemaphoreType.DMA,
        ],
    )
    def sgns_scores(centers_h, contexts_h, negs_h, inemb_h, outemb_h,
                    pos_h, neg_h, cidx, oidx, nidx, vcb, uob, neb,
                    posv, negv, semA, semB):
        wid = lax.axis_index("s") * _NC + lax.axis_index("c")
        lanes = lax.iota(jnp.int32, _L)
        sems = (semA, semB)

        pltpu.sync_copy(centers_h.at[pl.ds(wid * PW, PW)], cidx)
        pltpu.sync_copy(contexts_h.at[pl.ds(wid * PW, PW)], oidx)
        pltpu.sync_copy(negs_h.at[pl.ds(wid * NPW, NPW)], nidx)

        def copies(cc, b):
            sem = sems[b]
            cps = [
                pltpu.make_async_copy(
                    inemb_h.at[cidx.at[pl.ds(cc * CH, CH)]], vcb.at[b], sem),
                pltpu.make_async_copy(
                    outemb_h.at[oidx.at[pl.ds(cc * CH, CH)]], uob.at[b], sem),
            ]
            for j in range(NSL):
                cps.append(pltpu.make_async_copy(
                    outemb_h.at[nidx.at[pl.ds(cc * NEG_CH + j * ISL, ISL)]],
                    neb.at[b].at[pl.ds(j * ISL, ISL)], sem))
            return cps

        def issue(cc, b):
            for cp in copies(cc, b):
                cp.start()

        def drain(cc, b):
            for cp in copies(cc, b):
                cp.wait()

        def compute(cc, b):
            vc, uo, ne = vcb.at[b], uob.at[b], neb.at[b]
            nrows0 = lanes * _K
            zero = jnp.zeros((_L,), jnp.float32)

            def dblock(i, accs):
                pos_acc, negaccs = accs
                negaccs = list(negaccs)
                dbase = i * _DB
                for dd in range(_DB):
                    dcol = jnp.broadcast_to(dbase + dd, (_L,))
                    vcd = plsc.load_gather(vc, [lanes, dcol])
                    uod = plsc.load_gather(uo, [lanes, dcol])
                    pos_acc = pos_acc + vcd * uod
                    for k in range(_K):
                        nk = plsc.load_gather(ne, [nrows0 + k, dcol])
                        negaccs[k] = negaccs[k] + nk * vcd
                return pos_acc, tuple(negaccs)

            pos_acc, negaccs = lax.fori_loop(
                0, _D // _DB, dblock, (zero, (zero,) * _K))
            posv[pl.ds(cc * CH, CH)] = pos_acc
            for k in range(_K):
                negv[pl.ds(cc * NEG_CH + k * CH, CH)] = negaccs[k]

        issue(0, 0)
        issue(1, 1)

        def chunk_body(c, _):
            for b in range(2):
                cc = c * 2 + b
                drain(cc, b)
                compute(cc, b)

                @pl.when(cc + 2 < NCH)
                def _():
                    issue(cc + 2, b)
            return 0

        lax.fori_loop(0, NCH // 2, chunk_body, 0)

        pltpu.sync_copy(posv, pos_h.at[pl.ds(wid * PW, PW)])
        pltpu.sync_copy(negv, neg_h.at[pl.ds(wid * NPW, NPW)])

    return sgns_scores


@functools.lru_cache(maxsize=None)
def _make_tc_loss(B):
    def body(pos_ref, neg_ref, out_ref):
        p = pos_ref[...]
        n = neg_ref[...]
        lp = -jnp.log(jax.nn.sigmoid(p) + 1e-10)
        ln = -jnp.log(jax.nn.sigmoid(-n) + 1e-10)
        out_ref[...] = ((jnp.sum(lp) + jnp.sum(ln)) / B).reshape(1, 1)

    return pl.pallas_call(
        body,
        out_shape=jax.ShapeDtypeStruct((1, 1), jnp.float32),
    )


def kernel(centers, contexts, negs, in_embed, out_embed):
    B = centers.shape[0]
    c = centers.reshape(-1).astype(jnp.int32)
    o = contexts.reshape(-1).astype(jnp.int32)
    n = negs.reshape(-1).astype(jnp.int32)
    pos_s, neg_s = _make_sc_scores(B)(c, o, n, in_embed, out_embed)
    loss = _make_tc_loss(B)(
        pos_s.reshape(B // 128, 128), neg_s.reshape(B * _K // 128, 128))
    return loss[0, 0]
